# baseline (speedup 1.0000x reference)
import functools

import jax
import jax.numpy as jnp
from jax import lax
from jax.experimental import pallas as pl
from jax.experimental.pallas import tpu as pltpu
from jax.experimental.pallas import tpu_sc as plsc

B = 16384
L = 50
D = 32
DF = D + 1
NUM_WORKERS = 32
SPW = B // NUM_WORKERS
G = 16
NUM_GROUPS = SPW // G


def _fm_body(x_hbm, emb_hbm, bias_hbm, out_hbm,
             idx0, idx1, q0, q1, bias_v, res_v, sem0, sem1):
    wid = lax.axis_index("s") * 2 + lax.axis_index("c")
    iota = lax.broadcasted_iota(jnp.int32, (16,), 0)
    zero = jnp.zeros((16,), jnp.float32)

    pltpu.sync_copy(bias_hbm, bias_v)
    bias_vec = bias_v[...]

    slots = ((idx0, q0, sem0), (idx1, q1, sem1))

    def issue(g, slot):
        idx_v, q_v, sem = slot
        s0 = wid * SPW + g * G
        pltpu.sync_copy(x_hbm.at[pl.ds(s0, G)], idx_v)
        for s in range(G):
            pltpu.async_copy(emb_hbm.at[idx_v.at[s]], q_v.at[s], sem)

    def drain(slot):
        idx_v, q_v, sem = slot
        for s in range(G):
            pltpu.make_async_copy(
                emb_hbm.at[idx_v.at[s]], q_v.at[s], sem).wait()

    def compute(g, slot):
        idx_v, q_v, sem = slot
        dvecs = [jnp.full((16,), d, jnp.int32) for d in range(DF)]

        def l_body(l, lc):
            z, s2p, sl = lc
            lv = jnp.full((16,), l, jnp.int32)
            sl = sl + plsc.load_gather(q_v, [iota, lv, dvecs[D]])
            z = list(z)
            s2p = list(s2p)
            for d in range(D):
                a = plsc.load_gather(q_v, [iota, lv, dvecs[d]])
                z[d] = z[d] + a
                s2p[d % 4] = s2p[d % 4] + a * a
            return (tuple(z), tuple(s2p), sl)

        init = (tuple([zero] * D), (zero,) * 4, zero)
        z, s2p, sl = lax.fori_loop(0, L, l_body, init)

        s2 = (s2p[0] + s2p[1]) + (s2p[2] + s2p[3])
        z2p = [zero] * 4
        for d in range(D):
            z2p[d % 4] = z2p[d % 4] + z[d] * z[d]
        z2 = (z2p[0] + z2p[1]) + (z2p[2] + z2p[3])

        logit = sl + 0.5 * (s2 - z2) + bias_vec
        sig = 1.0 / (1.0 + jnp.exp(-logit))
        res_v[pl.ds(g * G, G)] = sig

    issue(0, slots[0])

    def t_body(t, carry):
        g0 = t * 2
        issue(g0 + 1, slots[1])
        drain(slots[0])
        compute(g0, slots[0])

        @pl.when(t < NUM_GROUPS // 2 - 1)
        def _():
            issue(g0 + 2, slots[0])

        drain(slots[1])
        compute(g0 + 1, slots[1])
        return carry

    lax.fori_loop(0, NUM_GROUPS // 2, t_body, 0)
    pltpu.sync_copy(res_v, out_hbm.at[pl.ds(wid * SPW, SPW)])


_fm_kernel = functools.partial(
    pl.kernel,
    mesh=plsc.VectorSubcoreMesh(core_axis_name="c", subcore_axis_name="s"),
    out_type=jax.ShapeDtypeStruct((B,), jnp.float32),
    compiler_params=pltpu.CompilerParams(
        needs_layout_passes=False, use_tc_tiling_on_sc=False),
    scratch_types=[
        pltpu.VMEM((G, L), jnp.int32),
        pltpu.VMEM((G, L), jnp.int32),
        pltpu.VMEM((G, L, DF), jnp.float32),
        pltpu.VMEM((G, L, DF), jnp.float32),
        pltpu.VMEM((16,), jnp.float32),
        pltpu.VMEM((SPW,), jnp.float32),
        pltpu.SemaphoreType.DMA,
        pltpu.SemaphoreType.DMA,
    ],
)(_fm_body)


@jax.jit
def kernel(X, embL, embQ, bias):
    Xi = X.astype(jnp.int32)
    emb = jnp.concatenate([embQ, embL], axis=1)
    bias16 = jnp.broadcast_to(bias.reshape(()), (16,))
    return _fm_kernel(Xi, emb, bias16)


# speedup vs baseline: 1.8923x; 1.8923x over previous
import functools

import jax
import jax.numpy as jnp
from jax import lax
from jax.experimental import pallas as pl
from jax.experimental.pallas import tpu as pltpu
from jax.experimental.pallas import tpu_sc as plsc

B = 16384
L = 50
D = 32
N = 1000000
NUM_WORKERS = 32
SPW = B // NUM_WORKERS
G = 16
NUM_GROUPS = SPW // G


def _fm_body(x_hbm, embl_hbm, embq_hbm, bias_hbm, out_hbm,
             idx0, idx1, q0, q1, el0, el1, bias_v, res_v, sem0, sem1):
    wid = lax.axis_index("s") * 2 + lax.axis_index("c")
    iota = lax.broadcasted_iota(jnp.int32, (16,), 0)
    zero = jnp.zeros((16,), jnp.float32)

    pltpu.sync_copy(bias_hbm, bias_v)
    bias_vec = bias_v[...]

    slots = ((idx0, q0, el0, sem0), (idx1, q1, el1, sem1))

    _GL = G * L
    _CH = [(c * 128, min(128, _GL - c * 128)) for c in range(-(-_GL // 128))]

    def issue(g, slot):
        idx_v, q_v, el_v, sem = slot
        f0 = (wid * SPW + g * G) * L
        pltpu.sync_copy(x_hbm.at[pl.ds(f0, _GL)], idx_v)
        for (o, n) in _CH:
            isl = idx_v.at[pl.ds(o, n)]
            pltpu.async_copy(embq_hbm.at[isl], q_v.at[pl.ds(o, n)], sem)
            pltpu.async_copy(embl_hbm.at[isl], el_v.at[pl.ds(o, n)], sem)

    def drain(slot):
        idx_v, q_v, el_v, sem = slot
        for (o, n) in _CH:
            isl = idx_v.at[pl.ds(o, n)]
            pltpu.make_async_copy(
                embq_hbm.at[isl], q_v.at[pl.ds(o, n)], sem).wait()
            pltpu.make_async_copy(
                embl_hbm.at[isl], el_v.at[pl.ds(o, n)], sem).wait()

    iota50 = iota * L

    def compute(g, slot):
        idx_v, q_v, el_v, sem = slot
        dvecs = [jnp.full((16,), d, jnp.int32) for d in range(D)]

        def l_body(l, lc):
            z, s2p, sl = lc
            lv = jnp.full((16,), l, jnp.int32)
            rv = iota50 + lv
            sl = sl + plsc.load_gather(el_v, [rv])
            z = list(z)
            s2p = list(s2p)
            for d in range(D):
                a = plsc.load_gather(q_v, [rv, dvecs[d]])
                z[d] = z[d] + a
                s2p[d % 4] = s2p[d % 4] + a * a
            return (tuple(z), tuple(s2p), sl)

        init = (tuple([zero] * D), (zero,) * 4, zero)
        z, s2p, sl = lax.fori_loop(0, L, l_body, init)

        s2 = (s2p[0] + s2p[1]) + (s2p[2] + s2p[3])
        z2p = [zero] * 4
        for d in range(D):
            z2p[d % 4] = z2p[d % 4] + z[d] * z[d]
        z2 = (z2p[0] + z2p[1]) + (z2p[2] + z2p[3])

        logit = sl + 0.5 * (s2 - z2) + bias_vec
        sig = 1.0 / (1.0 + jnp.exp(-logit))
        res_v[pl.ds(g * G, G)] = sig

    issue(0, slots[0])

    def t_body(t, carry):
        g0 = t * 2
        issue(g0 + 1, slots[1])
        drain(slots[0])
        compute(g0, slots[0])

        @pl.when(t < NUM_GROUPS // 2 - 1)
        def _():
            issue(g0 + 2, slots[0])

        drain(slots[1])
        compute(g0 + 1, slots[1])
        return carry

    lax.fori_loop(0, NUM_GROUPS // 2, t_body, 0)
    pltpu.sync_copy(res_v, out_hbm.at[pl.ds(wid * SPW, SPW)])


_fm_kernel = functools.partial(
    pl.kernel,
    mesh=plsc.VectorSubcoreMesh(core_axis_name="c", subcore_axis_name="s"),
    out_type=jax.ShapeDtypeStruct((B,), jnp.float32),
    compiler_params=pltpu.CompilerParams(
        needs_layout_passes=False, use_tc_tiling_on_sc=False),
    scratch_types=[
        pltpu.VMEM((G * L,), jnp.int32),
        pltpu.VMEM((G * L,), jnp.int32),
        pltpu.VMEM((G * L, D), jnp.float32),
        pltpu.VMEM((G * L, D), jnp.float32),
        pltpu.VMEM((G * L,), jnp.float32),
        pltpu.VMEM((G * L,), jnp.float32),
        pltpu.VMEM((16,), jnp.float32),
        pltpu.VMEM((SPW,), jnp.float32),
        pltpu.SemaphoreType.DMA,
        pltpu.SemaphoreType.DMA,
    ],
)(_fm_body)


@jax.jit
def kernel(X, embL, embQ, bias):
    Xi = X.reshape((-1,)).astype(jnp.int32)
    embL1 = embL.reshape((-1,))
    bias16 = jnp.broadcast_to(bias.reshape(()), (16,))
    return _fm_kernel(Xi, embL1, embQ, bias16)


# speedup vs baseline: 1.9109x; 1.0098x over previous
import functools

import jax
import jax.numpy as jnp
from jax import lax
from jax.experimental import pallas as pl
from jax.experimental.pallas import tpu as pltpu
from jax.experimental.pallas import tpu_sc as plsc

B = 16384
L = 50
D = 32
N = 1000000
NUM_WORKERS = 32
SPW = B // NUM_WORKERS
G = 32
NUM_GROUPS = SPW // G


def _fm_body(x_hbm, embl_hbm, embq_hbm, bias_hbm, out_hbm,
             idx0, idx1, q0, q1, el0, el1, bias_v, res_v, sem0, sem1):
    wid = lax.axis_index("s") * 2 + lax.axis_index("c")
    iota = lax.broadcasted_iota(jnp.int32, (16,), 0)
    zero = jnp.zeros((16,), jnp.float32)

    pltpu.sync_copy(bias_hbm, bias_v)
    bias_vec = bias_v[...]

    slots = ((idx0, q0, el0, sem0), (idx1, q1, el1, sem1))

    _GL = G * L
    _CH = [(c * 128, min(128, _GL - c * 128)) for c in range(-(-_GL // 128))]

    def issue(g, slot):
        idx_v, q_v, el_v, sem = slot
        f0 = (wid * SPW + g * G) * L
        pltpu.sync_copy(x_hbm.at[pl.ds(f0, _GL)], idx_v)
        for (o, n) in _CH:
            isl = idx_v.at[pl.ds(o, n)]
            pltpu.async_copy(embq_hbm.at[isl], q_v.at[pl.ds(o, n)], sem)
            pltpu.async_copy(embl_hbm.at[isl], el_v.at[pl.ds(o, n)], sem)

    def drain(slot):
        idx_v, q_v, el_v, sem = slot
        for (o, n) in _CH:
            isl = idx_v.at[pl.ds(o, n)]
            pltpu.make_async_copy(
                embq_hbm.at[isl], q_v.at[pl.ds(o, n)], sem).wait()
            pltpu.make_async_copy(
                embl_hbm.at[isl], el_v.at[pl.ds(o, n)], sem).wait()

    def compute(g, slot):
        idx_v, q_v, el_v, sem = slot
        dvecs = [jnp.full((16,), d, jnp.int32) for d in range(D)]

        for h in range(G // 16):
            iota50 = (iota + h * 16) * L

            def l_body(l, lc):
                z, s2p, sl = lc
                lv = jnp.full((16,), l, jnp.int32)
                rv = iota50 + lv
                sl = sl + plsc.load_gather(el_v, [rv])
                z = list(z)
                s2p = list(s2p)
                for d in range(D):
                    a = plsc.load_gather(q_v, [rv, dvecs[d]])
                    z[d] = z[d] + a
                    s2p[d % 4] = s2p[d % 4] + a * a
                return (tuple(z), tuple(s2p), sl)

            init = (tuple([zero] * D), (zero,) * 4, zero)
            z, s2p, sl = lax.fori_loop(0, L, l_body, init)

            s2 = (s2p[0] + s2p[1]) + (s2p[2] + s2p[3])
            z2p = [zero] * 4
            for d in range(D):
                z2p[d % 4] = z2p[d % 4] + z[d] * z[d]
            z2 = (z2p[0] + z2p[1]) + (z2p[2] + z2p[3])

            logit = sl + 0.5 * (s2 - z2) + bias_vec
            sig = 1.0 / (1.0 + jnp.exp(-logit))
            res_v[pl.ds(g * G + h * 16, 16)] = sig

    issue(0, slots[0])

    def t_body(t, carry):
        g0 = t * 2
        issue(g0 + 1, slots[1])
        drain(slots[0])
        compute(g0, slots[0])

        @pl.when(t < NUM_GROUPS // 2 - 1)
        def _():
            issue(g0 + 2, slots[0])

        drain(slots[1])
        compute(g0 + 1, slots[1])
        return carry

    lax.fori_loop(0, NUM_GROUPS // 2, t_body, 0)
    pltpu.sync_copy(res_v, out_hbm.at[pl.ds(wid * SPW, SPW)])


_fm_kernel = functools.partial(
    pl.kernel,
    mesh=plsc.VectorSubcoreMesh(core_axis_name="c", subcore_axis_name="s"),
    out_type=jax.ShapeDtypeStruct((B,), jnp.float32),
    compiler_params=pltpu.CompilerParams(
        needs_layout_passes=False, use_tc_tiling_on_sc=False),
    scratch_types=[
        pltpu.VMEM((G * L,), jnp.int32),
        pltpu.VMEM((G * L,), jnp.int32),
        pltpu.VMEM((G * L, D), jnp.float32),
        pltpu.VMEM((G * L, D), jnp.float32),
        pltpu.VMEM((G * L,), jnp.float32),
        pltpu.VMEM((G * L,), jnp.float32),
        pltpu.VMEM((16,), jnp.float32),
        pltpu.VMEM((SPW,), jnp.float32),
        pltpu.SemaphoreType.DMA,
        pltpu.SemaphoreType.DMA,
    ],
)(_fm_body)


@jax.jit
def kernel(X, embL, embQ, bias):
    Xi = X.reshape((-1,)).astype(jnp.int32)
    embL1 = embL.reshape((-1,))
    bias16 = jnp.broadcast_to(bias.reshape(()), (16,))
    return _fm_kernel(Xi, embL1, embQ, bias16)
